# baseline (speedup 1.0000x reference)
import jax
import jax.numpy as jnp
from jax.experimental import pallas as pl


def _one_batch(ms, ns):
    f32 = jnp.float32
    q = ms[0:4]
    q = q * jax.lax.rsqrt(jnp.sum(q * q, axis=0, keepdims=True) + 1e-30)
    qx, qy, qz, qw = q[0:1], q[1:2], q[2:3], q[3:4]
    R11 = 1 - 2 * (qy * qy + qz * qz)
    R12 = 2 * (qx * qy - qz * qw)
    R13 = 2 * (qx * qz + qy * qw)
    R21 = 2 * (qx * qy + qz * qw)
    R22 = 1 - 2 * (qx * qx + qz * qz)
    R23 = 2 * (qy * qz - qx * qw)
    R31 = 2 * (qx * qz - qy * qw)
    R32 = 2 * (qy * qz + qx * qw)
    R33 = 1 - 2 * (qx * qx + qy * qy)

    pts = ms[7:10]
    c = pts + ms[4:7]
    c1, c2, c3 = c[0:1], c[1:2], c[2:3]
    u1 = R11 * c1 + R12 * c2 + R13 * c3
    u2 = R21 * c1 + R22 * c2 + R23 * c3
    u3 = R31 * c1 + R32 * c2 + R33 * c3
    bm = c1 * c1 + c2 * c2 + c3 * c3
    ones_m = jnp.ones_like(bm)
    W = jnp.concatenate(
        [ones_m, bm, 2 * u1, 2 * u2, 2 * u3, -2 * c1, -2 * c2, -2 * c3,
         -2 * R11, -2 * R12, -2 * R13,
         -2 * R21, -2 * R22, -2 * R23,
         -2 * R31, -2 * R32, -2 * R33], axis=0)

    s1, s2, s3 = ns[0:1], ns[1:2], ns[2:3]
    t1, t2, t3 = ns[3:4], ns[4:5], ns[5:6]
    a_n = (s1 * s1 + s2 * s2 + s3 * s3
           + t1 * t1 + t2 * t2 + t3 * t3)
    ones_n = jnp.ones_like(a_n)
    F = jnp.concatenate(
        [a_n, ones_n, s1, s2, s3, t1, t2, t3,
         s1 * t1, s1 * t2, s1 * t3,
         s2 * t1, s2 * t2, s2 * t3,
         s3 * t1, s3 * t2, s3 * t3], axis=0)

    d2 = jax.lax.dot_general(
        F.astype(jnp.bfloat16), W.astype(jnp.bfloat16),
        (((0,), (0,)), ((), ())),
        preferred_element_type=f32)
    dc = jnp.maximum(d2, 1e-24)
    d = dc * jax.lax.rsqrt(dc)
    dis_h = jnp.mean(d, axis=0, keepdims=True)

    pc = jnp.maximum(ms[10:11], 1e-6)
    dispc_sum = jnp.sum(dis_h * pc, keepdims=True).reshape(1, 1)
    log_sum = jnp.sum(jnp.log(pc), keepdims=True).reshape(1, 1)

    m_iota = jax.lax.broadcasted_iota(jnp.int32, pc.shape, 1)
    maxv = jnp.max(pc)
    which = jnp.min(jnp.where(pc == maxv, m_iota, pc.shape[1]))
    onehot = (m_iota == which).astype(f32)

    dis_best = jnp.sum(dis_h * onehot, keepdims=True).reshape(1, 1)

    rb11 = jnp.sum(R11 * onehot)
    rb12 = jnp.sum(R12 * onehot)
    rb13 = jnp.sum(R13 * onehot)
    rb21 = jnp.sum(R21 * onehot)
    rb22 = jnp.sum(R22 * onehot)
    rb23 = jnp.sum(R23 * onehot)
    rb31 = jnp.sum(R31 * onehot)
    rb32 = jnp.sum(R32 * onehot)
    rb33 = jnp.sum(R33 * onehot)
    tb1 = jnp.sum(c1 * onehot)
    tb2 = jnp.sum(c2 * onehot)
    tb3 = jnp.sum(c3 * onehot)

    p1 = pts[0:1] - tb1
    p2 = pts[1:2] - tb2
    p3 = pts[2:3] - tb3
    np_rows = jnp.concatenate(
        [p1 * rb11 + p2 * rb21 + p3 * rb31,
         p1 * rb12 + p2 * rb22 + p3 * rb32,
         p1 * rb13 + p2 * rb23 + p3 * rb33], axis=0)

    g1 = t1 - tb1
    g2 = t2 - tb2
    g3 = t3 - tb3
    nt_rows = jnp.concatenate(
        [g1 * rb11 + g2 * rb21 + g3 * rb31,
         g1 * rb12 + g2 * rb22 + g3 * rb32,
         g1 * rb13 + g2 * rb23 + g3 * rb33], axis=0)

    return (jnp.concatenate([dispc_sum, log_sum, dis_best], axis=1),
            jnp.transpose(np_rows, (1, 0)), jnp.transpose(nt_rows, (1, 0)))


def _loss_body(ms_ref, ns_ref, sums_ref, np_ref, nt_ref):
    B = ms_ref.shape[0]
    acc = None
    for b in range(B):
        sums_b, np_b, nt_b = _one_batch(ms_ref[b], ns_ref[b])
        np_ref[b] = np_b
        nt_ref[b] = nt_b
        acc = sums_b if acc is None else acc + sums_b
    sums_ref[...] = acc


def kernel(pred_r, pred_t, pred_c, target, model_points, idx, points, w,
           refine, interpret=False):
    del idx, refine
    B, M, _ = pred_r.shape
    N = model_points.shape[1]
    f32 = jnp.float32

    ms = jnp.transpose(
        jnp.concatenate([pred_r, pred_t, points, pred_c[:, :, None]], axis=2),
        (0, 2, 1))
    ns = jnp.transpose(
        jnp.concatenate([model_points, target], axis=2),
        (0, 2, 1))
    sums, new_points, new_target = pl.pallas_call(
        _loss_body,
        grid=(1,),
        in_specs=[
            pl.BlockSpec((B, 11, M), lambda i: (0, 0, 0)),
            pl.BlockSpec((B, 6, N), lambda i: (0, 0, 0)),
        ],
        out_specs=(
            pl.BlockSpec((1, 3), lambda i: (0, 0)),
            pl.BlockSpec((B, M, 3), lambda i: (0, 0, 0)),
            pl.BlockSpec((B, N, 3), lambda i: (0, 0, 0)),
        ),
        out_shape=(
            jax.ShapeDtypeStruct((1, 3), f32),
            jax.ShapeDtypeStruct((B, M, 3), f32),
            jax.ShapeDtypeStruct((B, N, 3), f32),
        ),
        interpret=interpret,
    )(ms, ns)

    loss = (sums[0, 0] - w * sums[0, 1]) / (B * M)
    dis_best = sums[0, 2] / B
    return (loss, dis_best, new_points, new_target)


# speedup vs baseline: 1.2088x; 1.2088x over previous
import jax
import jax.numpy as jnp
from jax.experimental import pallas as pl


def _one_batch(ms, ns, w):
    f32 = jnp.float32
    q = ms[0:4]
    q = q * jax.lax.rsqrt(jnp.sum(q * q, axis=0, keepdims=True) + 1e-30)
    qx, qy, qz, qw = q[0:1], q[1:2], q[2:3], q[3:4]
    R11 = 1 - 2 * (qy * qy + qz * qz)
    R12 = 2 * (qx * qy - qz * qw)
    R13 = 2 * (qx * qz + qy * qw)
    R21 = 2 * (qx * qy + qz * qw)
    R22 = 1 - 2 * (qx * qx + qz * qz)
    R23 = 2 * (qy * qz - qx * qw)
    R31 = 2 * (qx * qz - qy * qw)
    R32 = 2 * (qy * qz + qx * qw)
    R33 = 1 - 2 * (qx * qx + qy * qy)

    pts = ms[7:10]
    c = pts + ms[4:7]
    c1, c2, c3 = c[0:1], c[1:2], c[2:3]
    u1 = R11 * c1 + R12 * c2 + R13 * c3
    u2 = R21 * c1 + R22 * c2 + R23 * c3
    u3 = R31 * c1 + R32 * c2 + R33 * c3
    bm = c1 * c1 + c2 * c2 + c3 * c3
    ones_m = jnp.ones_like(bm)
    W = jnp.concatenate(
        [ones_m, bm, 2 * u1, 2 * u2, 2 * u3, -2 * c1, -2 * c2, -2 * c3,
         -2 * R11, -2 * R12, -2 * R13,
         -2 * R21, -2 * R22, -2 * R23,
         -2 * R31, -2 * R32, -2 * R33], axis=0)

    s1, s2, s3 = ns[0:1], ns[1:2], ns[2:3]
    t1, t2, t3 = ns[3:4], ns[4:5], ns[5:6]
    a_n = (s1 * s1 + s2 * s2 + s3 * s3
           + t1 * t1 + t2 * t2 + t3 * t3)
    ones_n = jnp.ones_like(a_n)
    F = jnp.concatenate(
        [a_n, ones_n, s1, s2, s3, t1, t2, t3,
         s1 * t1, s1 * t2, s1 * t3,
         s2 * t1, s2 * t2, s2 * t3,
         s3 * t1, s3 * t2, s3 * t3], axis=0)

    d2 = jax.lax.dot_general(
        F.astype(jnp.bfloat16), W.astype(jnp.bfloat16),
        (((0,), (0,)), ((), ())),
        preferred_element_type=f32)
    dc = jnp.maximum(d2, 1e-24)
    d = dc * jax.lax.rsqrt(dc)
    dis_h = jnp.mean(d, axis=0, keepdims=True)

    pc = jnp.maximum(ms[10:11], 1e-6)
    loss_sum = jnp.sum(dis_h * pc - w * jnp.log(pc),
                       keepdims=True).reshape(1, 1)

    m_iota = jax.lax.broadcasted_iota(jnp.int32, pc.shape, 1)
    maxv = jnp.max(pc)
    which = jnp.min(jnp.where(pc == maxv, m_iota, pc.shape[1]))
    onehot = (m_iota == which).astype(f32)

    dis_best = jnp.sum(dis_h * onehot, keepdims=True).reshape(1, 1)

    rb11 = jnp.sum(R11 * onehot)
    rb12 = jnp.sum(R12 * onehot)
    rb13 = jnp.sum(R13 * onehot)
    rb21 = jnp.sum(R21 * onehot)
    rb22 = jnp.sum(R22 * onehot)
    rb23 = jnp.sum(R23 * onehot)
    rb31 = jnp.sum(R31 * onehot)
    rb32 = jnp.sum(R32 * onehot)
    rb33 = jnp.sum(R33 * onehot)
    tb1 = jnp.sum(c1 * onehot)
    tb2 = jnp.sum(c2 * onehot)
    tb3 = jnp.sum(c3 * onehot)

    p1 = pts[0:1] - tb1
    p2 = pts[1:2] - tb2
    p3 = pts[2:3] - tb3
    np_rows = jnp.concatenate(
        [p1 * rb11 + p2 * rb21 + p3 * rb31,
         p1 * rb12 + p2 * rb22 + p3 * rb32,
         p1 * rb13 + p2 * rb23 + p3 * rb33], axis=0)

    g1 = t1 - tb1
    g2 = t2 - tb2
    g3 = t3 - tb3
    nt_rows = jnp.concatenate(
        [g1 * rb11 + g2 * rb21 + g3 * rb31,
         g1 * rb12 + g2 * rb22 + g3 * rb32,
         g1 * rb13 + g2 * rb23 + g3 * rb33], axis=0)

    return (loss_sum, dis_best,
            jnp.transpose(np_rows, (1, 0)), jnp.transpose(nt_rows, (1, 0)))


def _loss_body(w_ref, ms_ref, ns_ref, loss_ref, disb_ref, np_ref, nt_ref):
    w = w_ref[0, 0]
    B = ms_ref.shape[0]
    M = ms_ref.shape[2]
    loss_acc = None
    disb_acc = None
    for b in range(B):
        loss_b, disb_b, np_b, nt_b = _one_batch(ms_ref[b], ns_ref[b], w)
        np_ref[b] = np_b
        nt_ref[b] = nt_b
        loss_acc = loss_b if loss_acc is None else loss_acc + loss_b
        disb_acc = disb_b if disb_acc is None else disb_acc + disb_b
    loss_ref[...] = loss_acc * (1.0 / (B * M))
    disb_ref[...] = disb_acc * (1.0 / B)


def kernel(pred_r, pred_t, pred_c, target, model_points, idx, points, w,
           refine, interpret=False):
    del idx, refine
    B, M, _ = pred_r.shape
    N = model_points.shape[1]
    f32 = jnp.float32

    ms = jnp.transpose(
        jnp.concatenate([pred_r, pred_t, points, pred_c[:, :, None]], axis=2),
        (0, 2, 1))
    ns = jnp.transpose(
        jnp.concatenate([model_points, target], axis=2),
        (0, 2, 1))
    wArr = jnp.full((1, 1), w, f32)

    loss2d, disb2d, new_points, new_target = pl.pallas_call(
        _loss_body,
        grid=(1,),
        in_specs=[
            pl.BlockSpec((1, 1), lambda i: (0, 0)),
            pl.BlockSpec((B, 11, M), lambda i: (0, 0, 0)),
            pl.BlockSpec((B, 6, N), lambda i: (0, 0, 0)),
        ],
        out_specs=(
            pl.BlockSpec((1, 1), lambda i: (0, 0)),
            pl.BlockSpec((1, 1), lambda i: (0, 0)),
            pl.BlockSpec((B, M, 3), lambda i: (0, 0, 0)),
            pl.BlockSpec((B, N, 3), lambda i: (0, 0, 0)),
        ),
        out_shape=(
            jax.ShapeDtypeStruct((1, 1), f32),
            jax.ShapeDtypeStruct((1, 1), f32),
            jax.ShapeDtypeStruct((B, M, 3), f32),
            jax.ShapeDtypeStruct((B, N, 3), f32),
        ),
        interpret=interpret,
    )(wArr, ms, ns)

    return (loss2d[0, 0], disb2d[0, 0], new_points, new_target)
